# baseline (speedup 1.0000x reference)
import functools

import jax
import jax.numpy as jnp
from jax import lax
from jax.experimental import pallas as pl
from jax.experimental.pallas import tpu as pltpu
from jax.experimental.pallas import tpu_sc as plsc

BATCH = 4096
N_LOG = 64 * 64
N_PP = 8 * 24
P = 1858
P_PAD = 1920
L = 16
NSLOT = (P + L - 1) // L
LAST_OFF = P - L
R = 8
NA = NSLOT * L


def _sc_policy_gather(logits, pp_logits, idx_pad):
    info = plsc.get_sparse_core_info()
    nc, ns = info.num_cores, info.num_subcores
    nw = nc * ns
    rows_per_w = BATCH // nw
    n_chunks = rows_per_w // R

    mesh = plsc.VectorSubcoreMesh(core_axis_name="c", subcore_axis_name="s")

    @functools.partial(
        pl.kernel,
        mesh=mesh,
        out_type=jax.ShapeDtypeStruct((BATCH, P), jnp.float32),
        compiler_params=pltpu.CompilerParams(
            needs_layout_passes=False, use_tc_tiling_on_sc=False),
        scratch_types=[
            pltpu.VMEM((P_PAD,), jnp.int32),
            pltpu.VMEM((NA,), jnp.int32),
            pltpu.VMEM((NA,), jnp.int32),
            pltpu.VMEM((NA,), jnp.int32),
            pltpu.VMEM((NA,), jnp.int32),
            pltpu.VMEM((NA,), jnp.int32),
            pltpu.VMEM((R * 64, 64), jnp.float32),
            pltpu.VMEM((R * 64, 64), jnp.float32),
            pltpu.VMEM((R * 8, 24), jnp.float32),
            pltpu.VMEM((R * 8, 24), jnp.float32),
            pltpu.VMEM((R, P), jnp.float32),
            pltpu.VMEM((R, P), jnp.float32),
            pltpu.SemaphoreType.DMA,
            pltpu.SemaphoreType.DMA,
            pltpu.SemaphoreType.DMA,
            pltpu.SemaphoreType.DMA,
        ],
    )
    def k(log3_hbm, pp3_hbm, idx_hbm, out_hbm,
          idx_v, iraw_v, lmaj_v, lmin_v, pmaj_v, pmin_v,
          log_a, log_b, pp_a, pp_b, out_a, out_b,
          sin_a, sin_b, sout_a, sout_b):
        wid = lax.axis_index("s") * nc + lax.axis_index("c")
        base = wid * rows_per_w

        pltpu.sync_copy(idx_hbm, idx_v)

        def prep(kslot, carry):
            off = jnp.where(kslot == NSLOT - 1, LAST_OFF, kslot * L)
            iv = idx_v[pl.ds(off, L)]
            li = jnp.minimum(iv, N_LOG - 1)
            pi = jnp.clip(iv - N_LOG, 0, N_PP - 1)
            pq = pi // 24
            o = kslot * L
            iraw_v[pl.ds(o, L)] = iv
            lmaj_v[pl.ds(o, L)] = li >> 6
            lmin_v[pl.ds(o, L)] = li & 63
            pmaj_v[pl.ds(o, L)] = pq
            pmin_v[pl.ds(o, L)] = pi - pq * 24
            return carry

        lax.fori_loop(0, NSLOT, prep, 0)

        def issue_in(c, log_st, pp_st, sem):
            r0 = base + c * R
            for r in range(R):
                pltpu.async_copy(
                    log3_hbm.at[r0 + r], log_st.at[pl.ds(r * 64, 64)], sem)
                pltpu.async_copy(
                    pp3_hbm.at[r0 + r], pp_st.at[pl.ds(r * 8, 8)], sem)

        def wait_in(log_st, pp_st, sem):
            for r in range(R):
                pltpu.make_async_copy(
                    log3_hbm.at[0], log_st.at[pl.ds(r * 64, 64)], sem).wait()
                pltpu.make_async_copy(
                    pp3_hbm.at[0], pp_st.at[pl.ds(r * 8, 8)], sem).wait()

        def issue_out(c, out_v, sem):
            r0 = base + c * R
            pltpu.async_copy(out_v, out_hbm.at[pl.ds(r0, R)], sem)

        def wait_out(out_v, sem):
            pltpu.make_async_copy(out_v, out_hbm.at[pl.ds(0, R)], sem).wait()

        def compute(log_st, pp_st, out_v):
            def slot(kk, carry):
                o = kk * L
                off = jnp.where(kk == NSLOT - 1, LAST_OFF, kk * L)
                iv = iraw_v[pl.ds(o, L)]
                lmaj = lmaj_v[pl.ds(o, L)]
                lmin = lmin_v[pl.ds(o, L)]
                pmaj = pmaj_v[pl.ds(o, L)]
                pmin = pmin_v[pl.ds(o, L)]
                msk = iv < N_LOG
                for r in range(R):
                    vlog = plsc.load_gather(
                        log_st,
                        [lmaj + jnp.full((L,), r * 64, jnp.int32), lmin])
                    vpp = plsc.load_gather(
                        pp_st,
                        [pmaj + jnp.full((L,), r * 8, jnp.int32), pmin])
                    out_v[r, pl.ds(off, L)] = jnp.where(msk, vlog, vpp)
                return carry

            lax.fori_loop(0, NSLOT, slot, 0)

        issue_in(0, log_a, pp_a, sin_a)
        issue_in(1, log_b, pp_b, sin_b)

        def pair(g, carry):
            c0 = 2 * g
            wait_in(log_a, pp_a, sin_a)

            @pl.when(g > 0)
            def _():
                wait_out(out_a, sout_a)

            compute(log_a, pp_a, out_a)
            issue_out(c0, out_a, sout_a)

            @pl.when(g < n_chunks // 2 - 1)
            def _():
                issue_in(c0 + 2, log_a, pp_a, sin_a)

            wait_in(log_b, pp_b, sin_b)

            @pl.when(g > 0)
            def _():
                wait_out(out_b, sout_b)

            compute(log_b, pp_b, out_b)
            issue_out(c0 + 1, out_b, sout_b)

            @pl.when(g < n_chunks // 2 - 1)
            def _():
                issue_in(c0 + 3, log_b, pp_b, sin_b)

            return carry

        lax.fori_loop(0, n_chunks // 2, pair, 0)
        wait_out(out_a, sout_a)
        wait_out(out_b, sout_b)

    return k(logits, pp_logits, idx_pad)


def kernel(logits, pp_logits, idx):
    idx_pad = jnp.pad(idx, (0, P_PAD - P))
    return _sc_policy_gather(logits, pp_logits, idx_pad)
